# baseline (speedup 1.0000x reference)
import functools

import jax
import jax.numpy as jnp
from jax import lax
from jax.experimental import pallas as pl
from jax.experimental.pallas import tpu as pltpu
from jax.experimental.pallas import tpu_sc as plsc

VOCAB = 100000
DIM = 64
B = 4096
L = 200
LP = 208
PAD_IDX = 0

NC = 2
NS = 16
NW = NC * NS
NV = B // NW
LH = LP // 2

BV = 32


NBUF = 4
DIMW = 80


def _sc_gather_body(idx_hbm, emb_hbm, out_hbm,
                    idx_v, rows, sems):
    wid = lax.axis_index("s") * NC + lax.axis_index("c")
    base = wid * NV
    pltpu.sync_copy(idx_hbm.at[pl.ds(base * LP, NV * LP)], idx_v)

    def start(h, b):
        pltpu.make_async_copy(emb_hbm.at[idx_v.at[pl.ds(h * LH, LH)]],
                              rows.at[b], sems.at[b]).start()

    def wait(h, b):
        pltpu.make_async_copy(emb_hbm.at[idx_v.at[pl.ds(h * LH, LH)]],
                              rows.at[b], sems.at[b]).wait()

    for b in range(NBUF):
        start(b, b)

    def outer(i, carry):
        h0 = NBUF * i
        for b in range(NBUF):
            h = h0 + b
            wait(h, b)

            @pl.when(h + NBUF < 2 * NV)
            def _():
                start(h + NBUF, b)
        return carry

    lax.fori_loop(0, 2 * NV // NBUF, outer, 0)
    pltpu.sync_copy(rows.at[0], out_hbm.at[pl.ds(base * LP, LH)])


_sc_gather = functools.partial(
    pl.kernel,
    out_type=jax.ShapeDtypeStruct((B * LP, DIMW), jnp.float32),
    mesh=plsc.VectorSubcoreMesh(core_axis_name="c", subcore_axis_name="s"),
    compiler_params=pltpu.CompilerParams(use_tc_tiling_on_sc=False),
    scratch_types=[
        pltpu.VMEM((NV * LP,), jnp.int32),
        pltpu.VMEM((NBUF, LH, DIMW), jnp.float32),
        pltpu.SemaphoreType.DMA((NBUF,)),
    ],
)(_sc_gather_body)


def _tc_pool_body(z_ref, idx_ref, out_ref):
    z = z_ref[:, :DIM]
    idx = idx_ref[...]
    x2 = jnp.sum(z * z, axis=-1)
    gamma = 2.0 / jnp.maximum(1.0 - x2, 1e-15)
    m = (idx != PAD_IDX).astype(jnp.float32)
    wg = m * gamma.reshape(BV, LP)
    z3 = z.reshape(BV, LP, DIM)
    nom = jnp.sum(wg[..., None] * z3, axis=1)
    den = jnp.sum(wg - m, axis=1, keepdims=True)
    cnt = jnp.sum(m, axis=1, keepdims=True)

    ms = jnp.where(cnt == 0.0, 1.0, cnt)
    nom = nom / ms
    den = den / ms
    den = jnp.where(jnp.abs(den) < 1e-10, 1e-10, den)
    two_mean = nom / den
    tn2 = jnp.sum(two_mean * two_mean, axis=-1, keepdims=True)
    tn = jnp.sqrt(jnp.clip(tn2, 1e-15, None))
    arg = jnp.minimum(tn, 1.0 - 1e-5)
    half = arg / (1.0 + jnp.sqrt(jnp.maximum(1.0 - arg * arg, 0.0)))
    mid = half * two_mean / tn
    mn2 = jnp.sum(mid * mid, axis=-1, keepdims=True)
    mn = jnp.sqrt(jnp.clip(mn2, 1e-15, None))
    marg = jnp.minimum(mn, 1.0 - 1e-5)
    at = 0.5 * jnp.log((1.0 + marg) / (1.0 - marg))
    tangent = at * mid / mn
    out_ref[...] = jnp.where(cnt == 0.0, 0.0, tangent)


def kernel(flat_visits, emb):
    idx_p = jnp.pad(flat_visits, ((0, 0), (0, LP - L)),
                    constant_values=PAD_IDX)
    w = jnp.pad(emb, ((0, 0), (0, DIMW - DIM)))
    gathered = _sc_gather(idx_p.reshape(B * LP), w)
    out = pl.pallas_call(
        _tc_pool_body,
        grid=(B // BV,),
        in_specs=[
            pl.BlockSpec((BV * LP, DIMW), lambda i: (i, 0)),
            pl.BlockSpec((BV, LP), lambda i: (i, 0)),
        ],
        out_specs=pl.BlockSpec((BV, DIM), lambda i: (i, 0)),
        out_shape=jax.ShapeDtypeStruct((B, DIM), jnp.float32),
    )(gathered, idx_p)
    return out


# speedup vs baseline: 1.8823x; 1.8823x over previous
import functools

import jax
import jax.numpy as jnp
from jax import lax
from jax.experimental import pallas as pl
from jax.experimental.pallas import tpu as pltpu
from jax.experimental.pallas import tpu_sc as plsc

VOCAB = 100000
DIM = 64
B = 4096
L = 200
LP = 208
PAD_IDX = 0

NC = 2
NS = 16
NW = NC * NS
NV = B // NW
LH = LP // 2
VC = 2048
GTF = 100352


def _sc_pool_body(idx_hbm, emb_hbm, gt_hbm, nom_hbm, aux_hbm,
                  idx_v, rows0, rows1, gb0, gb1, nom_acc, aux_acc,
                  sem0, sem1):
    wid = lax.axis_index("s") * NC + lax.axis_index("c")
    base = wid * NV
    pltpu.sync_copy(idx_hbm.at[pl.ds(base * LP, NV * LP)], idx_v)

    rows = (rows0, rows1)
    gbs = (gb0, gb1)
    sems = (sem0, sem1)

    def start(v, b):
        pltpu.make_async_copy(emb_hbm.at[idx_v.at[pl.ds(v * LP, LH)]],
                              rows[b].at[pl.ds(0, LH)], sems[b]).start()
        pltpu.make_async_copy(emb_hbm.at[idx_v.at[pl.ds(v * LP + LH, LH)]],
                              rows[b].at[pl.ds(LH, LH)], sems[b]).start()
        pltpu.make_async_copy(gt_hbm.at[idx_v.at[pl.ds(v * LP, LH)]],
                              gbs[b].at[pl.ds(0, LH)], sems[b]).start()
        pltpu.make_async_copy(gt_hbm.at[idx_v.at[pl.ds(v * LP + LH, LH)]],
                              gbs[b].at[pl.ds(LH, LH)], sems[b]).start()

    def wait(v, b):
        pltpu.make_async_copy(emb_hbm.at[idx_v.at[pl.ds(v * LP, LH)]],
                              rows[b].at[pl.ds(0, LH)], sems[b]).wait()
        pltpu.make_async_copy(emb_hbm.at[idx_v.at[pl.ds(v * LP + LH, LH)]],
                              rows[b].at[pl.ds(LH, LH)], sems[b]).wait()
        pltpu.make_async_copy(gt_hbm.at[idx_v.at[pl.ds(v * LP, LH)]],
                              gbs[b].at[pl.ds(0, LH)], sems[b]).wait()
        pltpu.make_async_copy(gt_hbm.at[idx_v.at[pl.ds(v * LP + LH, LH)]],
                              gbs[b].at[pl.ds(LH, LH)], sems[b]).wait()

    start(0, 0)
    start(1, 1)

    _dnums = lax.GatherDimensionNumbers(
        offset_dims=(), collapsed_slice_dims=(0,), start_index_map=(0,))

    def _permute(x, p):
        return lax.gather(x, p.reshape(16, 1), _dnums, (1,),
                          mode=lax.GatherScatterMode.PROMISE_IN_BOUNDS)

    def compute(v, b):
        r = rows[b]
        gb = gbs[b]

        def group_body(gi, carry):
            n0, n1, n2, n3, dv, cv = carry
            iv = idx_v[pl.ds(v * LP + 16 * gi, 16)]
            vf16 = jnp.where(iv != PAD_IDX, 1.0, 0.0).astype(jnp.float32)
            cv = cv + vf16
            gv = gb[pl.ds(16 * gi, 16)]
            dv = dv + gv
            for j in range(16):
                jj = 16 * gi + j
                g = _permute(gv, jnp.full((16,), j, jnp.int32))
                z0 = r[jj, pl.ds(0, 16)]
                z1 = r[jj, pl.ds(16, 16)]
                z2 = r[jj, pl.ds(32, 16)]
                z3 = r[jj, pl.ds(48, 16)]
                n0 = n0 + g * z0
                n1 = n1 + g * z1
                n2 = n2 + g * z2
                n3 = n3 + g * z3
            return (n0, n1, n2, n3, dv, cv)

        z16 = jnp.zeros((16,), jnp.float32)
        n0, n1, n2, n3, dv, cv = lax.fori_loop(
            0, LP // 16, group_body, (z16, z16, z16, z16, z16, z16))
        nom_acc[v, pl.ds(0, 16)] = n0
        nom_acc[v, pl.ds(16, 16)] = n1
        nom_acc[v, pl.ds(32, 16)] = n2
        nom_acc[v, pl.ds(48, 16)] = n3
        aux_acc[v, pl.ds(0, 16)] = dv
        aux_acc[v, pl.ds(16, 16)] = cv

    def outer(i, carry):
        v0 = 2 * i
        for b in range(2):
            v = v0 + b
            wait(v, b)
            compute(v, b)

            @pl.when(v + 2 < NV)
            def _():
                start(v + 2, b)
        return carry

    lax.fori_loop(0, NV // 2, outer, 0)

    pltpu.sync_copy(nom_acc, nom_hbm.at[pl.ds(base, NV)])
    pltpu.sync_copy(aux_acc, aux_hbm.at[pl.ds(base, NV)])


_sc_pool = functools.partial(
    pl.kernel,
    out_type=[
        jax.ShapeDtypeStruct((B, DIM), jnp.float32),
        jax.ShapeDtypeStruct((B, 32), jnp.float32),
    ],
    mesh=plsc.VectorSubcoreMesh(core_axis_name="c", subcore_axis_name="s"),
    compiler_params=pltpu.CompilerParams(use_tc_tiling_on_sc=False),
    scratch_types=[
        pltpu.VMEM((NV * LP,), jnp.int32),
        pltpu.VMEM((LP, DIM), jnp.float32),
        pltpu.VMEM((LP, DIM), jnp.float32),
        pltpu.VMEM((LP,), jnp.float32),
        pltpu.VMEM((LP,), jnp.float32),
        pltpu.VMEM((NV, DIM), jnp.float32),
        pltpu.VMEM((NV, 32), jnp.float32),
        pltpu.SemaphoreType.DMA,
        pltpu.SemaphoreType.DMA,
    ],
)(_sc_pool_body)


def _fin_body(nom_ref, aux_ref, emb0_ref, out_ref):
    nom_all = nom_ref[...]
    gsum = jnp.sum(aux_ref[:, 0:16], axis=-1, keepdims=True)
    cnt = jnp.sum(aux_ref[:, 16:32], axis=-1, keepdims=True)
    emb0 = emb0_ref[...]
    e0sq = jnp.sum(emb0 * emb0, axis=-1, keepdims=True)
    gamma0 = 2.0 / jnp.maximum(1.0 - e0sq, 1e-15)
    npad = LP - cnt
    nom_raw = nom_all - (npad * gamma0) * emb0
    den_raw = gsum - npad * gamma0 - cnt
    ms = jnp.where(cnt == 0.0, 1.0, cnt)
    nom = nom_raw / ms
    den = den_raw / ms
    den = jnp.where(jnp.abs(den) < 1e-10, 1e-10, den)
    two_mean = nom / den
    tn2 = jnp.sum(two_mean * two_mean, axis=-1, keepdims=True)
    tn = jnp.sqrt(jnp.clip(tn2, 1e-15, None))
    arg = jnp.minimum(tn, 1.0 - 1e-5)
    half = arg / (1.0 + jnp.sqrt(jnp.maximum(1.0 - arg * arg, 0.0)))
    mid = half * two_mean / tn
    mn2 = jnp.sum(mid * mid, axis=-1, keepdims=True)
    mn = jnp.sqrt(jnp.clip(mn2, 1e-15, None))
    marg = jnp.minimum(mn, 1.0 - 1e-5)
    at = 0.5 * jnp.log((1.0 + marg) / (1.0 - marg))
    tangent = at * mid / mn
    out_ref[...] = jnp.where(cnt == 0.0, 0.0, tangent)


def _gt_body(emb_ref, gt_ref):
    e = emb_ref[...]
    x2 = jnp.sum(e * e, axis=-1, keepdims=True)
    g = 2.0 / jnp.maximum(1.0 - x2, 1e-15)
    gt_ref[...] = g.reshape(VC)


def kernel(flat_visits, emb):
    idx_p = jnp.pad(flat_visits, ((0, 0), (0, LP - L)),
                    constant_values=PAD_IDX).reshape(B * LP)
    gt = pl.pallas_call(
        _gt_body,
        grid=(GTF // VC,),
        in_specs=[pl.BlockSpec((VC, DIM), lambda i: (i, 0))],
        out_specs=pl.BlockSpec((VC,), lambda i: (i,)),
        out_shape=jax.ShapeDtypeStruct((GTF,), jnp.float32),
    )(emb)
    nom_raw, aux = _sc_pool(idx_p, emb, gt)
    out = pl.pallas_call(
        _fin_body,
        out_shape=jax.ShapeDtypeStruct((B, DIM), jnp.float32),
    )(nom_raw, aux, emb[PAD_IDX:PAD_IDX + 1])
    return out


# speedup vs baseline: 2.0459x; 1.0869x over previous
import functools

import jax
import jax.numpy as jnp
from jax import lax
from jax.experimental import pallas as pl
from jax.experimental.pallas import tpu as pltpu
from jax.experimental.pallas import tpu_sc as plsc

VOCAB = 100000
DIM = 64
B = 4096
L = 200
LP = 208
PAD_IDX = 0

NC = 2
NS = 16
NW = NC * NS
NV = B // NW
LH = LP // 2


def _sc_pool_body(idx_hbm, emb_hbm, nom_hbm, aux_hbm,
                  idx_v, rows0, rows1, nom_acc, aux_acc, sem0, sem1):
    wid = lax.axis_index("s") * NC + lax.axis_index("c")
    base = wid * NV
    pltpu.sync_copy(idx_hbm.at[pl.ds(base * LP, NV * LP)], idx_v)

    rows = (rows0, rows1)
    sems = (sem0, sem1)

    def start(v, b):
        pltpu.make_async_copy(emb_hbm.at[idx_v.at[pl.ds(v * LP, LH)]],
                              rows[b].at[pl.ds(0, LH)], sems[b]).start()
        pltpu.make_async_copy(emb_hbm.at[idx_v.at[pl.ds(v * LP + LH, LH)]],
                              rows[b].at[pl.ds(LH, LH)], sems[b]).start()

    def wait(v, b):
        pltpu.make_async_copy(emb_hbm.at[idx_v.at[pl.ds(v * LP, LH)]],
                              rows[b].at[pl.ds(0, LH)], sems[b]).wait()
        pltpu.make_async_copy(emb_hbm.at[idx_v.at[pl.ds(v * LP + LH, LH)]],
                              rows[b].at[pl.ds(LH, LH)], sems[b]).wait()

    start(0, 0)
    start(1, 1)

    lanes = lax.iota(jnp.int32, 16)
    perms = [(lanes ^ k).reshape(16, 1) for k in (1, 2, 4, 8)]
    _dnums = lax.GatherDimensionNumbers(
        offset_dims=(), collapsed_slice_dims=(0,), start_index_map=(0,))

    def _permute(x, p):
        return lax.gather(x, p, _dnums, (1,),
                          mode=lax.GatherScatterMode.PROMISE_IN_BOUNDS)

    def compute(v, b):
        r = rows[b]

        def group_body(gi, carry):
            n0, n1, n2, n3, dv, cv = carry
            iv = idx_v[pl.ds(v * LP + 16 * gi, 16)]
            vf16 = jnp.where(iv != PAD_IDX, 1.0, 0.0).astype(jnp.float32)
            cv = cv + vf16
            for j in range(16):
                jj = 16 * gi + j
                z0 = r[jj, pl.ds(0, 16)]
                z1 = r[jj, pl.ds(16, 16)]
                z2 = r[jj, pl.ds(32, 16)]
                z3 = r[jj, pl.ds(48, 16)]
                s = z0 * z0 + z1 * z1 + z2 * z2 + z3 * z3
                for p in perms:
                    s = s + _permute(s, p)
                g = 2.0 / jnp.maximum(1.0 - s, 1e-15)
                n0 = n0 + g * z0
                n1 = n1 + g * z1
                n2 = n2 + g * z2
                n3 = n3 + g * z3
                dv = dv + g
            return (n0, n1, n2, n3, dv, cv)

        z16 = jnp.zeros((16,), jnp.float32)
        n0, n1, n2, n3, dv, cv = lax.fori_loop(
            0, LP // 16, group_body, (z16, z16, z16, z16, z16, z16))
        nom_acc[v, pl.ds(0, 16)] = n0
        nom_acc[v, pl.ds(16, 16)] = n1
        nom_acc[v, pl.ds(32, 16)] = n2
        nom_acc[v, pl.ds(48, 16)] = n3
        aux_acc[v, pl.ds(0, 16)] = dv
        aux_acc[v, pl.ds(16, 16)] = cv

    def outer(i, carry):
        v0 = 2 * i
        for b in range(2):
            v = v0 + b
            wait(v, b)
            compute(v, b)

            @pl.when(v + 2 < NV)
            def _():
                start(v + 2, b)
        return carry

    lax.fori_loop(0, NV // 2, outer, 0)

    pltpu.sync_copy(nom_acc, nom_hbm.at[pl.ds(base, NV)])
    pltpu.sync_copy(aux_acc, aux_hbm.at[pl.ds(base, NV)])


_sc_pool = functools.partial(
    pl.kernel,
    out_type=[
        jax.ShapeDtypeStruct((B, DIM), jnp.float32),
        jax.ShapeDtypeStruct((B, 32), jnp.float32),
    ],
    mesh=plsc.VectorSubcoreMesh(core_axis_name="c", subcore_axis_name="s"),
    compiler_params=pltpu.CompilerParams(use_tc_tiling_on_sc=False),
    scratch_types=[
        pltpu.VMEM((NV * LP,), jnp.int32),
        pltpu.VMEM((LP, DIM), jnp.float32),
        pltpu.VMEM((LP, DIM), jnp.float32),
        pltpu.VMEM((NV, DIM), jnp.float32),
        pltpu.VMEM((NV, 32), jnp.float32),
        pltpu.SemaphoreType.DMA,
        pltpu.SemaphoreType.DMA,
    ],
)(_sc_pool_body)


def _fin_body(nom_ref, aux_ref, emb0_ref, out_ref):
    nom_all = nom_ref[...]
    gsum = aux_ref[:, 0:1]
    cnt = jnp.sum(aux_ref[:, 16:32], axis=-1, keepdims=True)
    emb0 = emb0_ref[...]
    e0sq = jnp.sum(emb0 * emb0, axis=-1, keepdims=True)
    gamma0 = 2.0 / jnp.maximum(1.0 - e0sq, 1e-15)
    npad = LP - cnt
    nom_raw = nom_all - (npad * gamma0) * emb0
    den_raw = gsum - npad * gamma0 - cnt
    ms = jnp.where(cnt == 0.0, 1.0, cnt)
    nom = nom_raw / ms
    den = den_raw / ms
    den = jnp.where(jnp.abs(den) < 1e-10, 1e-10, den)
    two_mean = nom / den
    tn2 = jnp.sum(two_mean * two_mean, axis=-1, keepdims=True)
    tn = jnp.sqrt(jnp.clip(tn2, 1e-15, None))
    arg = jnp.minimum(tn, 1.0 - 1e-5)
    half = arg / (1.0 + jnp.sqrt(jnp.maximum(1.0 - arg * arg, 0.0)))
    mid = half * two_mean / tn
    mn2 = jnp.sum(mid * mid, axis=-1, keepdims=True)
    mn = jnp.sqrt(jnp.clip(mn2, 1e-15, None))
    marg = jnp.minimum(mn, 1.0 - 1e-5)
    at = 0.5 * jnp.log((1.0 + marg) / (1.0 - marg))
    tangent = at * mid / mn
    out_ref[...] = jnp.where(cnt == 0.0, 0.0, tangent)


def kernel(flat_visits, emb):
    idx_p = jnp.pad(flat_visits, ((0, 0), (0, LP - L)),
                    constant_values=PAD_IDX).reshape(B * LP)
    nom_raw, aux = _sc_pool(idx_p, emb)
    out = pl.pallas_call(
        _fin_body,
        out_shape=jax.ShapeDtypeStruct((B, DIM), jnp.float32),
    )(nom_raw, aux, emb[PAD_IDX:PAD_IDX + 1])
    return out
